# baseline (speedup 1.0000x reference)
import functools

import jax
import jax.numpy as jnp
from jax import lax
from jax.experimental import pallas as pl
from jax.experimental.pallas import tpu as pltpu
from jax.experimental.pallas import tpu_sc as plsc

N_CLASSES = 1000000
EMBED_DIM = 32
BATCH = 16384
FIELDS = 100

NUM_CORES = 2
NUM_SUBCORES = 16
NW = NUM_CORES * NUM_SUBCORES
BPW = BATCH // NW
LANES = 16
ETILES = EMBED_DIM // 8
BTILES = BPW // 128
NPAIR = FIELDS // 2
GSPLIT = 4
GCH = BPW // GSPLIT

_mesh = plsc.VectorSubcoreMesh(core_axis_name="c", subcore_axis_name="s")


@functools.partial(
    pl.kernel,
    mesh=_mesh,
    out_type=jax.ShapeDtypeStruct(
        (FIELDS, ETILES, BATCH // 128, 8, 128), jnp.float32),
    scratch_types=[
        pltpu.VMEM((BPW,), jnp.int32),
        pltpu.VMEM((BPW,), jnp.int32),
        pltpu.VMEM((BPW, EMBED_DIM), jnp.float32),
        pltpu.VMEM((BPW, EMBED_DIM), jnp.float32),
        pltpu.VMEM((ETILES, BTILES, 8, 128), jnp.float32),
        pltpu.VMEM((ETILES, BTILES, 8, 128), jnp.float32),
        pltpu.SemaphoreType.DMA,
        pltpu.SemaphoreType.DMA,
        pltpu.SemaphoreType.DMA,
    ],
    compiler_params=pltpu.CompilerParams(
        use_tc_tiling_on_sc=False, needs_layout_passes=False),
)
def _emb_lookup(idx_hbm, table_hbm, out_hbm,
                idx_a, idx_b, rows_a, rows_b, t_a, t_b,
                sem_i, sem_g, sem_o):
    wid = lax.axis_index("s") * NUM_CORES + lax.axis_index("c")
    base_b = wid * BPW
    bt0 = wid * BTILES
    ar16 = jnp.arange(LANES, dtype=jnp.int32)

    def idx_src(f):
        return idx_hbm.at[pl.ds(f * BATCH + base_b, BPW)]

    def transpose_into(rows_v, t_v):
        def g_body(g, c):
            q = g // 8
            j = g % 8
            row_ids = g * LANES + ar16
            vals = [
                plsc.load_gather(
                    rows_v, [row_ids, jnp.full((LANES,), e, jnp.int32)])
                for e in range(EMBED_DIM)
            ]
            for e in range(EMBED_DIM):
                t_v[e // 8, q, e % 8, pl.ds(j * LANES, LANES)] = (
                    jnp.abs(vals[e]))
            return c
        lax.fori_loop(0, BPW // LANES, g_body, 0)

    def fire_out(t_v, f):
        for e_t in range(ETILES):
            pltpu.async_copy(t_v.at[e_t],
                             out_hbm.at[f, e_t, pl.ds(bt0, BTILES)], sem_o)

    def drain_out():
        for e_t in range(ETILES):
            pltpu.make_async_copy(
                t_a.at[e_t], out_hbm.at[0, e_t, pl.ds(bt0, BTILES)],
                sem_o).wait()

    def fire_gather(idx_v, rows_v):
        for s in range(GSPLIT):
            pltpu.async_copy(
                table_hbm.at[idx_v.at[pl.ds(s * GCH, GCH)]],
                rows_v.at[pl.ds(s * GCH, GCH)], sem_g)

    def drain_gather(rows_v):
        for s in range(GSPLIT):
            pltpu.make_async_copy(
                table_hbm.at[pl.ds(0, GCH)],
                rows_v.at[pl.ds(s * GCH, GCH)], sem_g).wait()

    def drain_idx(idx_v):
        pltpu.make_async_copy(idx_src(0), idx_v, sem_i).wait()

    pltpu.sync_copy(idx_src(0), idx_a)
    fire_gather(idx_a, rows_a)
    pltpu.async_copy(idx_src(1), idx_b, sem_i)

    def pair_body(k, carry):
        f0 = 2 * k
        f1 = f0 + 1

        with jax.named_scope("drainG"):
            drain_gather(rows_a)

        with jax.named_scope("idxfire"):
            drain_idx(idx_b)
            fire_gather(idx_b, rows_b)

        @pl.when(f0 + 2 < FIELDS)
        def _():
            pltpu.async_copy(idx_src(f0 + 2), idx_a, sem_i)

        @pl.when(k >= 1)
        def _():
            drain_out()

        with jax.named_scope("xpose"):
            transpose_into(rows_a, t_a)
        with jax.named_scope("outs"):
            fire_out(t_a, f0)

        drain_gather(rows_b)

        @pl.when(f1 + 1 < FIELDS)
        def _():
            drain_idx(idx_a)
            fire_gather(idx_a, rows_a)

        @pl.when(f1 + 2 < FIELDS)
        def _():
            pltpu.async_copy(idx_src(f1 + 2), idx_b, sem_i)

        @pl.when(k >= 1)
        def _():
            drain_out()

        transpose_into(rows_b, t_b)
        fire_out(t_b, f1)
        return carry

    lax.fori_loop(0, NPAIR, pair_body, 0)

    drain_out()
    drain_out()


def kernel(inputs, table):
    idx_t = inputs.T.reshape(-1).astype(jnp.int32)
    out5 = _emb_lookup(idx_t, table)
    return out5.transpose((2, 4, 0, 1, 3)).reshape(BATCH, FIELDS, EMBED_DIM)


# speedup vs baseline: 1.2772x; 1.2772x over previous
import functools

import jax
import jax.numpy as jnp
from jax import lax
from jax.experimental import pallas as pl
from jax.experimental.pallas import tpu as pltpu
from jax.experimental.pallas import tpu_sc as plsc

N_CLASSES = 1000000
EMBED_DIM = 32
BATCH = 16384
FIELDS = 100

NUM_CORES = 2
NUM_SUBCORES = 16
NW = NUM_CORES * NUM_SUBCORES
BPW = BATCH // NW
LANES = 16
ETILES = EMBED_DIM // 8
BTILES = BPW // 128
NPAIR = FIELDS // 2
GSPLIT = 1
GCH = BPW // GSPLIT

_mesh = plsc.VectorSubcoreMesh(core_axis_name="c", subcore_axis_name="s")


@functools.partial(
    pl.kernel,
    mesh=_mesh,
    out_type=jax.ShapeDtypeStruct(
        (FIELDS * ETILES * (BATCH // 128) * 8 * 128,), jnp.float32),
    scratch_types=[
        pltpu.VMEM((BPW,), jnp.int32),
        pltpu.VMEM((BPW,), jnp.int32),
        pltpu.VMEM((BPW, EMBED_DIM), jnp.float32),
        pltpu.VMEM((BPW, EMBED_DIM), jnp.float32),
        pltpu.VMEM((ETILES * BTILES * 8 * 128,), jnp.float32),
        pltpu.VMEM((ETILES * BTILES * 8 * 128,), jnp.float32),
        pltpu.SemaphoreType.DMA,
        pltpu.SemaphoreType.DMA,
        pltpu.SemaphoreType.DMA,
    ],
    compiler_params=pltpu.CompilerParams(
        use_tc_tiling_on_sc=False, needs_layout_passes=False),
)
def _emb_lookup(idx_hbm, table_hbm, out_hbm,
                idx_a, idx_b, rows_a, rows_b, t_a, t_b,
                sem_i, sem_g, sem_o):
    wid = lax.axis_index("s") * NUM_CORES + lax.axis_index("c")
    base_b = wid * BPW
    bt0 = wid * BTILES
    ar16 = jnp.arange(LANES, dtype=jnp.int32)

    def idx_src(f):
        return idx_hbm.at[pl.ds(f * BATCH + base_b, BPW)]

    def transpose_into(rows_v, t_v):
        def g_body(g, c):
            q = g // 8
            b_ids = g * LANES + ar16
            base_qb = q * 1024 + (g % 8) * LANES + ar16
            for e0 in range(EMBED_DIM):
                e_vec = jnp.bitwise_xor(ar16, e0)
                vals = plsc.load_gather(rows_v, [b_ids, e_vec])
                fe = ((e_vec & 24) << 9) | ((e_vec & 7) << 7)
                plsc.store_scatter(t_v, [base_qb + fe], jnp.abs(vals))
            return c
        lax.fori_loop(0, BPW // LANES, g_body, 0)

    def fire_out(t_v, f):
        for e_t in range(ETILES):
            off = ((f * ETILES + e_t) * (BATCH // 128) + bt0) * 1024
            pltpu.async_copy(t_v.at[pl.ds(e_t * 4096, 4096)],
                             out_hbm.at[pl.ds(off, 4096)], sem_o)

    def drain_out():
        for e_t in range(ETILES):
            pltpu.make_async_copy(
                t_a.at[pl.ds(e_t * 4096, 4096)],
                out_hbm.at[pl.ds(bt0 * 1024, 4096)], sem_o).wait()

    def fire_gather(idx_v, rows_v):
        for s in range(GSPLIT):
            pltpu.async_copy(
                table_hbm.at[idx_v.at[pl.ds(s * GCH, GCH)]],
                rows_v.at[pl.ds(s * GCH, GCH)], sem_g)

    def drain_gather(rows_v):
        for s in range(GSPLIT):
            pltpu.make_async_copy(
                table_hbm.at[pl.ds(0, GCH)],
                rows_v.at[pl.ds(s * GCH, GCH)], sem_g).wait()

    def drain_idx(idx_v):
        pltpu.make_async_copy(idx_src(0), idx_v, sem_i).wait()

    pltpu.sync_copy(idx_src(0), idx_a)
    fire_gather(idx_a, rows_a)
    pltpu.async_copy(idx_src(1), idx_b, sem_i)

    def pair_body(k, carry):
        f0 = 2 * k
        f1 = f0 + 1

        drain_gather(rows_a)

        drain_idx(idx_b)
        fire_gather(idx_b, rows_b)

        @pl.when(f0 + 2 < FIELDS)
        def _():
            pltpu.async_copy(idx_src(f0 + 2), idx_a, sem_i)

        @pl.when(k >= 1)
        def _():
            drain_out()

        transpose_into(rows_a, t_a)
        fire_out(t_a, f0)

        drain_gather(rows_b)

        @pl.when(f1 + 1 < FIELDS)
        def _():
            drain_idx(idx_a)
            fire_gather(idx_a, rows_a)

        @pl.when(f1 + 2 < FIELDS)
        def _():
            pltpu.async_copy(idx_src(f1 + 2), idx_b, sem_i)

        @pl.when(k >= 1)
        def _():
            drain_out()

        transpose_into(rows_b, t_b)
        fire_out(t_b, f1)
        return carry

    lax.fori_loop(0, NPAIR, pair_body, 0)

    drain_out()
    drain_out()


def kernel(inputs, table):
    idx_t = inputs.T.reshape(-1).astype(jnp.int32)
    out5 = _emb_lookup(idx_t, table).reshape(
        FIELDS, ETILES, BATCH // 128, 8, 128)
    return out5.transpose((2, 4, 0, 1, 3)).reshape(BATCH, FIELDS, EMBED_DIM)


# speedup vs baseline: 1.7941x; 1.4046x over previous
import functools

import jax
import jax.numpy as jnp
from jax import lax
from jax.experimental import pallas as pl
from jax.experimental.pallas import tpu as pltpu
from jax.experimental.pallas import tpu_sc as plsc

N_CLASSES = 1000000
EMBED_DIM = 32
BATCH = 16384
FIELDS = 100

NUM_CORES = 2
NUM_SUBCORES = 16
NW = NUM_CORES * NUM_SUBCORES
BPW = BATCH // NW
LANES = 16
ETILES = EMBED_DIM // 8
BTILES = BPW // 128
NPAIR = FIELDS // 2
GSPLIT = 1
GCH = BPW // GSPLIT

_mesh = plsc.VectorSubcoreMesh(core_axis_name="c", subcore_axis_name="s")


@functools.partial(
    pl.kernel,
    mesh=_mesh,
    out_type=jax.ShapeDtypeStruct(
        (FIELDS * ETILES * (BATCH // 128) * 8 * 128,), jnp.float32),
    scratch_types=[
        pltpu.VMEM((BPW,), jnp.int32),
        pltpu.VMEM((BPW,), jnp.int32),
        pltpu.VMEM((BPW, EMBED_DIM), jnp.float32),
        pltpu.VMEM((BPW, EMBED_DIM), jnp.float32),
        pltpu.VMEM((ETILES * BTILES * 8 * 128,), jnp.float32),
        pltpu.VMEM((ETILES * BTILES * 8 * 128,), jnp.float32),
        pltpu.SemaphoreType.DMA,
        pltpu.SemaphoreType.DMA,
        pltpu.SemaphoreType.DMA,
    ],
    compiler_params=pltpu.CompilerParams(
        use_tc_tiling_on_sc=False, needs_layout_passes=False),
)
def _emb_lookup(idx_hbm, table_hbm, out_hbm,
                idx_a, idx_b, rows_a, rows_b, t_a, t_b,
                sem_i, sem_g, sem_o):
    wid = lax.axis_index("s") * NUM_CORES + lax.axis_index("c")
    base_b = wid * BPW
    bt0 = wid * BTILES
    ar16 = jnp.arange(LANES, dtype=jnp.int32)

    def idx_src(f):
        return idx_hbm.at[pl.ds(f * BATCH + base_b, BPW)]

    fe0_vec = ((ar16 & 24) << 9) | ((ar16 & 7) << 7)

    def transpose_into(rows_v, t_v):
        def g_body(g, c):
            q = g // 8
            b_ids = g * LANES + ar16
            base_qb = q * 1024 + (g % 8) * LANES + ar16
            for blk in range(EMBED_DIM // 8):
                vals = [
                    plsc.load_gather(
                        rows_v, [b_ids, jnp.bitwise_xor(ar16, blk * 8 + i)])
                    for i in range(8)
                ]
                for i in range(8):
                    e0 = blk * 8 + i
                    fe0 = ((e0 & 24) << 9) | ((e0 & 7) << 7)
                    addr = base_qb | jnp.bitwise_xor(fe0_vec, fe0)
                    plsc.store_scatter(t_v, [addr], jnp.abs(vals[i]))
            return c
        lax.fori_loop(0, BPW // LANES, g_body, 0)

    def fire_out(t_v, f):
        for e_t in range(ETILES):
            off = ((f * ETILES + e_t) * (BATCH // 128) + bt0) * 1024
            pltpu.async_copy(t_v.at[pl.ds(e_t * 4096, 4096)],
                             out_hbm.at[pl.ds(off, 4096)], sem_o)

    def drain_out():
        for e_t in range(ETILES):
            pltpu.make_async_copy(
                t_a.at[pl.ds(e_t * 4096, 4096)],
                out_hbm.at[pl.ds(bt0 * 1024, 4096)], sem_o).wait()

    def fire_gather(idx_v, rows_v):
        for s in range(GSPLIT):
            pltpu.async_copy(
                table_hbm.at[idx_v.at[pl.ds(s * GCH, GCH)]],
                rows_v.at[pl.ds(s * GCH, GCH)], sem_g)

    def drain_gather(rows_v):
        for s in range(GSPLIT):
            pltpu.make_async_copy(
                table_hbm.at[pl.ds(0, GCH)],
                rows_v.at[pl.ds(s * GCH, GCH)], sem_g).wait()

    def drain_idx(idx_v):
        pltpu.make_async_copy(idx_src(0), idx_v, sem_i).wait()

    pltpu.sync_copy(idx_src(0), idx_a)
    fire_gather(idx_a, rows_a)
    pltpu.async_copy(idx_src(1), idx_b, sem_i)

    def pair_body(k, carry):
        f0 = 2 * k
        f1 = f0 + 1

        drain_gather(rows_a)

        drain_idx(idx_b)
        fire_gather(idx_b, rows_b)

        @pl.when(f0 + 2 < FIELDS)
        def _():
            pltpu.async_copy(idx_src(f0 + 2), idx_a, sem_i)

        @pl.when(k >= 1)
        def _():
            drain_out()

        transpose_into(rows_a, t_a)
        fire_out(t_a, f0)

        drain_gather(rows_b)

        @pl.when(f1 + 1 < FIELDS)
        def _():
            drain_idx(idx_a)
            fire_gather(idx_a, rows_a)

        @pl.when(f1 + 2 < FIELDS)
        def _():
            pltpu.async_copy(idx_src(f1 + 2), idx_b, sem_i)

        @pl.when(k >= 1)
        def _():
            drain_out()

        transpose_into(rows_b, t_b)
        fire_out(t_b, f1)
        return carry

    lax.fori_loop(0, NPAIR, pair_body, 0)

    drain_out()
    drain_out()


def kernel(inputs, table):
    idx_t = inputs.T.reshape(-1).astype(jnp.int32)
    out5 = _emb_lookup(idx_t, table).reshape(
        FIELDS, ETILES, BATCH // 128, 8, 128)
    return out5.transpose((2, 4, 0, 1, 3)).reshape(BATCH, FIELDS, EMBED_DIM)


# speedup vs baseline: 1.8112x; 1.0096x over previous
import functools

import jax
import jax.numpy as jnp
from jax import lax
from jax.experimental import pallas as pl
from jax.experimental.pallas import tpu as pltpu
from jax.experimental.pallas import tpu_sc as plsc

N_CLASSES = 1000000
EMBED_DIM = 32
BATCH = 16384
FIELDS = 100

NUM_CORES = 2
NUM_SUBCORES = 16
NW = NUM_CORES * NUM_SUBCORES
BPW = BATCH // NW
LANES = 16
ETILES = EMBED_DIM // 8
BTILES = BPW // 128
NBLK = FIELDS // 4

_mesh = plsc.VectorSubcoreMesh(core_axis_name="c", subcore_axis_name="s")


@functools.partial(
    pl.kernel,
    mesh=_mesh,
    out_type=jax.ShapeDtypeStruct(
        (FIELDS * ETILES * (BATCH // 128) * 8 * 128,), jnp.float32),
    scratch_types=[
        pltpu.VMEM((2, 4, BPW), jnp.int32),
        pltpu.VMEM((BPW, EMBED_DIM), jnp.float32),
        pltpu.VMEM((BPW, EMBED_DIM), jnp.float32),
        pltpu.VMEM((BPW, EMBED_DIM), jnp.float32),
        pltpu.VMEM((BPW, EMBED_DIM), jnp.float32),
        pltpu.VMEM((ETILES * BTILES * 8 * 128,), jnp.float32),
        pltpu.VMEM((ETILES * BTILES * 8 * 128,), jnp.float32),
        pltpu.SemaphoreType.DMA,
        pltpu.SemaphoreType.DMA,
        pltpu.SemaphoreType.DMA,
    ],
    compiler_params=pltpu.CompilerParams(
        use_tc_tiling_on_sc=False, needs_layout_passes=False),
)
def _emb_lookup(idx_hbm, table_hbm, out_hbm,
                idx_v, rows_0, rows_1, rows_2, rows_3, t_a, t_b,
                sem_i, sem_g, sem_o):
    wid = lax.axis_index("s") * NUM_CORES + lax.axis_index("c")
    base_b = wid * BPW
    bt0 = wid * BTILES
    ar16 = jnp.arange(LANES, dtype=jnp.int32)

    fe0_vec = ((ar16 & 24) << 9) | ((ar16 & 7) << 7)

    def idx_block_src(k):
        return idx_hbm.at[pl.ds(4 * k, 4), pl.ds(base_b, BPW)]

    def fire_idx(k, p):
        pltpu.async_copy(idx_block_src(k), idx_v.at[p], sem_i)

    def drain_idx(p):
        pltpu.make_async_copy(idx_block_src(0), idx_v.at[p], sem_i).wait()

    def fire_gather(p, i, rows_v):
        pltpu.async_copy(table_hbm.at[idx_v.at[p, i]], rows_v, sem_g)

    def drain_gather(rows_v):
        pltpu.make_async_copy(
            table_hbm.at[pl.ds(0, BPW)], rows_v, sem_g).wait()

    def transpose_into(rows_v, t_v):
        def g_body(g, c):
            q = g // 8
            b_ids = g * LANES + ar16
            base_qb = q * 1024 + (g % 8) * LANES + ar16
            for blk in range(EMBED_DIM // 8):
                vals = [
                    plsc.load_gather(
                        rows_v, [b_ids, jnp.bitwise_xor(ar16, blk * 8 + i)])
                    for i in range(8)
                ]
                for i in range(8):
                    e0 = blk * 8 + i
                    fe0 = ((e0 & 24) << 9) | ((e0 & 7) << 7)
                    addr = base_qb | jnp.bitwise_xor(fe0_vec, fe0)
                    plsc.store_scatter(t_v, [addr], jnp.abs(vals[i]))
            return c
        lax.fori_loop(0, BPW // LANES, g_body, 0)

    def fire_out(t_v, f):
        for e_t in range(ETILES):
            off = ((f * ETILES + e_t) * (BATCH // 128) + bt0) * 1024
            pltpu.async_copy(t_v.at[pl.ds(e_t * 4096, 4096)],
                             out_hbm.at[pl.ds(off, 4096)], sem_o)

    def drain_out():
        for e_t in range(ETILES):
            pltpu.make_async_copy(
                t_a.at[pl.ds(e_t * 4096, 4096)],
                out_hbm.at[pl.ds(bt0 * 1024, 4096)], sem_o).wait()

    pltpu.sync_copy(idx_block_src(0), idx_v.at[0])
    fire_gather(0, 0, rows_0)
    fire_gather(0, 1, rows_1)
    fire_gather(0, 2, rows_2)

    def blk_body(k, carry):
        p = k % 2
        f0 = 4 * k

        @pl.when(k + 1 < NBLK)
        def _():
            fire_idx(k + 1, 1 - p)
        drain_gather(rows_0)
        fire_gather(p, 3, rows_3)
        @pl.when(k >= 1)
        def _():
            drain_out()
        transpose_into(rows_0, t_a)
        fire_out(t_a, f0)

        @pl.when(k + 1 < NBLK)
        def _():
            drain_idx(1 - p)
        drain_gather(rows_1)
        @pl.when(k + 1 < NBLK)
        def _():
            fire_gather(1 - p, 0, rows_0)
        @pl.when(k >= 1)
        def _():
            drain_out()
        transpose_into(rows_1, t_b)
        fire_out(t_b, f0 + 1)

        drain_gather(rows_2)
        @pl.when(k + 1 < NBLK)
        def _():
            fire_gather(1 - p, 1, rows_1)
        drain_out()
        transpose_into(rows_2, t_a)
        fire_out(t_a, f0 + 2)

        drain_gather(rows_3)
        @pl.when(k + 1 < NBLK)
        def _():
            fire_gather(1 - p, 2, rows_2)
        drain_out()
        transpose_into(rows_3, t_b)
        fire_out(t_b, f0 + 3)
        return carry

    lax.fori_loop(0, NBLK, blk_body, 0)

    drain_out()
    drain_out()


def kernel(inputs, table):
    idx_t = inputs.T.astype(jnp.int32)
    out5 = _emb_lookup(idx_t, table).reshape(
        FIELDS, ETILES, BATCH // 128, 8, 128)
    return out5.transpose((2, 4, 0, 1, 3)).reshape(BATCH, FIELDS, EMBED_DIM)
